# Initial kernel scaffold; baseline (speedup 1.0000x reference)
import jax
import jax.numpy as jnp
from jax.experimental import pallas as pl


def kernel(feats, edge_index, W1, b1, W2, b2):
    raise NotImplementedError("write your pallas kernel here")


# speedup vs baseline: 3.3390x; 3.3390x over previous
import jax
import jax.numpy as jnp
from jax import lax
from jax.experimental import pallas as pl
from jax.experimental.pallas import tpu as pltpu
from jax.experimental.pallas import tpu_sc as plsc

N = 10000
D = 128
E = 320000

NC = 2
NS = 16
NW = NC * NS
NP = 10112
RPT = NP // NS
B = 128
NCHUNK = 80
EW = NCHUNK * B
EP = NW * EW
DG = 128
PAD_ROW = N + 16

_mesh = plsc.VectorSubcoreMesh(core_axis_name="c", subcore_axis_name="s")


def _fill(ref, val):
    nr, ncols = ref.shape
    v = jnp.full((16,), val, jnp.float32)
    def outer(i, _):
        for j in range(ncols // 16):
            ref[i, pl.ds(j * 16, 16)] = v
        return 0
    lax.fori_loop(0, nr, outer, 0)


def _zero_slab(zsrc, shared, rbase):
    for k in range(4):
        pltpu.sync_copy(zsrc.at[pl.ds(0, B)], shared.at[pl.ds(rbase + k * B, B)])
    pltpu.sync_copy(zsrc.at[pl.ds(0, RPT - 4 * B)],
                    shared.at[pl.ds(rbase + 4 * B, RPT - 4 * B)])


def _sc_deg_body(dst_hbm, deg_hbm, dstv, ones, dacc, sem):
    c = lax.axis_index("c")
    s = lax.axis_index("s")
    w = c * NS + s
    rbase = s * RPT
    _fill(ones, 0.0)
    _zero_slab(ones, dacc, rbase)
    _fill(ones, 1.0)
    plsc.subcore_barrier()
    ebase = w * EW
    def chunk(i, _):
        pltpu.sync_copy(dst_hbm.at[pl.ds(ebase + i * B, B)], dstv)
        pltpu.sync_copy(ones, dacc.at[dstv], add=True)
        return 0
    lax.fori_loop(0, NCHUNK, chunk, 0)
    plsc.subcore_barrier()
    pltpu.sync_copy(dacc.at[pl.ds(rbase, RPT)],
                    deg_hbm.at[pl.ds(c * NP + rbase, RPT)])


_sc_deg = pl.kernel(
    _sc_deg_body,
    out_type=[jax.ShapeDtypeStruct((NC * NP, DG), jnp.float32)],
    mesh=_mesh,
    scratch_types=[
        pltpu.VMEM((B,), jnp.int32),
        pltpu.VMEM((B, DG), jnp.float32),
        pltpu.VMEM_SHARED((NP, DG), jnp.float32),
        pltpu.SemaphoreType.DMA,
    ],
)


def _sc_scatter_body(y_hbm, src_hbm, dst_hbm, out_hbm,
                     srcv0, dstv0, rows0, srcv1, dstv1, rows1, acc, sem0, sem1):
    c = lax.axis_index("c")
    s = lax.axis_index("s")
    w = c * NS + s
    rbase = s * RPT
    _fill(rows0, 0.0)
    _zero_slab(rows0, acc, rbase)
    plsc.subcore_barrier()
    ebase = w * EW
    srcs = (srcv0, srcv1)
    dsts = (dstv0, dstv1)
    rows = (rows0, rows1)
    sems = (sem0, sem1)
    for b in range(2):
        base = ebase + b * B
        pltpu.sync_copy(src_hbm.at[pl.ds(base, B)], srcs[b])
        pltpu.sync_copy(dst_hbm.at[pl.ds(base, B)], dsts[b])
        pltpu.async_copy(y_hbm.at[srcs[b]], rows[b], sems[b])
    def body(i, _):
        for b in range(2):
            g = 2 * i + b
            pltpu.make_async_copy(y_hbm.at[srcs[b]], rows[b], sems[b]).wait()
            pltpu.sync_copy(rows[b], acc.at[dsts[b]], add=True)
            @pl.when(g + 2 < NCHUNK)
            def _():
                base = ebase + (g + 2) * B
                pltpu.sync_copy(src_hbm.at[pl.ds(base, B)], srcs[b])
                pltpu.sync_copy(dst_hbm.at[pl.ds(base, B)], dsts[b])
                pltpu.async_copy(y_hbm.at[srcs[b]], rows[b], sems[b])
        return 0
    lax.fori_loop(0, NCHUNK // 2, body, 0)
    plsc.subcore_barrier()
    pltpu.sync_copy(acc.at[pl.ds(rbase, RPT)],
                    out_hbm.at[pl.ds(c * NP + rbase, RPT)])


_sc_scatter = pl.kernel(
    _sc_scatter_body,
    out_type=[jax.ShapeDtypeStruct((NC * NP, D), jnp.float32)],
    mesh=_mesh,
    scratch_types=[
        pltpu.VMEM((B,), jnp.int32),
        pltpu.VMEM((B,), jnp.int32),
        pltpu.VMEM((B, D), jnp.float32),
        pltpu.VMEM((B,), jnp.int32),
        pltpu.VMEM((B,), jnp.int32),
        pltpu.VMEM((B, D), jnp.float32),
        pltpu.VMEM_SHARED((NP, D), jnp.float32),
        pltpu.SemaphoreType.DMA,
        pltpu.SemaphoreType.DMA,
    ],
)

RB = 1000


def _mm_body(x_ref, w_ref, o_ref):
    o_ref[...] = jnp.dot(x_ref[...], w_ref[...], preferred_element_type=jnp.float32)


def _matmul(x, W):
    return pl.pallas_call(
        _mm_body,
        grid=(N // RB,),
        in_specs=[pl.BlockSpec((RB, D), lambda i: (i, 0)),
                  pl.BlockSpec((D, D), lambda i: (0, 0))],
        out_specs=pl.BlockSpec((RB, D), lambda i: (i, 0)),
        out_shape=jax.ShapeDtypeStruct((N, D), jnp.float32),
    )(x, W)


def _combine1_body(p_ref, y_ref, d_ref, b_ref, w2_ref, h1_ref, y2_ref):
    ssum = p_ref[0] + p_ref[1]
    deg = d_ref[0, :, 0:1] + d_ref[1, :, 0:1]
    h = (ssum + y_ref[...]) / (deg + 1.0) + b_ref[...]
    h1 = jnp.maximum(h, 0.0)
    h1_ref[...] = h1
    y2_ref[...] = jnp.dot(h1, w2_ref[...], preferred_element_type=jnp.float32)


def _combine1(P, y1, Dg, b1, W2):
    return pl.pallas_call(
        _combine1_body,
        grid=(N // RB,),
        in_specs=[pl.BlockSpec((NC, RB, D), lambda i: (0, i, 0)),
                  pl.BlockSpec((RB, D), lambda i: (i, 0)),
                  pl.BlockSpec((NC, RB, DG), lambda i: (0, i, 0)),
                  pl.BlockSpec((1, D), lambda i: (0, 0)),
                  pl.BlockSpec((D, D), lambda i: (0, 0))],
        out_specs=[pl.BlockSpec((RB, D), lambda i: (i, 0)),
                   pl.BlockSpec((RB, D), lambda i: (i, 0))],
        out_shape=[jax.ShapeDtypeStruct((N, D), jnp.float32),
                   jax.ShapeDtypeStruct((N, D), jnp.float32)],
    )(P, y1, Dg, b1, W2)


def _final_body(q_ref, y_ref, d_ref, b_ref, h2_ref):
    ssum = q_ref[0] + q_ref[1]
    deg = d_ref[0, :, 0:1] + d_ref[1, :, 0:1]
    h2_ref[...] = (ssum + y_ref[...]) / (deg + 1.0) + b_ref[...]


def _final(Q, y2, Dg, b2):
    return pl.pallas_call(
        _final_body,
        grid=(N // RB,),
        in_specs=[pl.BlockSpec((NC, RB, D), lambda i: (0, i, 0)),
                  pl.BlockSpec((RB, D), lambda i: (i, 0)),
                  pl.BlockSpec((NC, RB, DG), lambda i: (0, i, 0)),
                  pl.BlockSpec((1, D), lambda i: (0, 0))],
        out_specs=pl.BlockSpec((RB, D), lambda i: (i, 0)),
        out_shape=jax.ShapeDtypeStruct((N, D), jnp.float32),
    )(Q, y2, Dg, b2)


def kernel(feats, edge_index, W1, b1, W2, b2):
    npad = EP - E
    src = jnp.concatenate([edge_index[0], jnp.zeros((npad,), jnp.int32)])
    dst = jnp.concatenate([edge_index[1], jnp.full((npad,), PAD_ROW, jnp.int32)])
    y1 = _matmul(feats, W1)
    Dg = _sc_deg(dst)[0].reshape(NC, NP, DG)
    P1 = _sc_scatter(y1, src, dst)[0].reshape(NC, NP, D)
    h1, y2 = _combine1(P1, y1, Dg, b1.reshape(1, D), W2)
    P2 = _sc_scatter(y2, src, dst)[0].reshape(NC, NP, D)
    h2 = _final(P2, y2, Dg, b2.reshape(1, D))
    return (h1, h2)
